# baseline (speedup 1.0000x reference)
import jax
import jax.numpy as jnp
from jax.experimental import pallas as pl

_N_EMBED = 8192
_E_DIM = 256
_BM = 256
_WINDOW = 2736


def _vq_body(z_ref, e_ref, out_ref, idx_ref, rss_ref):
    z = z_ref[...]
    e = e_ref[...]
    zsq = jnp.sum(z * z, axis=1, keepdims=True)
    esq = jnp.sum(e * e, axis=1)[None, :]
    dot = jax.lax.dot_general(z.astype(jnp.bfloat16), e.astype(jnp.bfloat16),
                              (((1,), (1,)), ((), ())),
                              preferred_element_type=jnp.float32)
    sim = (zsq + esq) - 2.0 * dot

    iota = jax.lax.broadcasted_iota(jnp.int32, sim.shape, 1)
    neg = jnp.float32(-jnp.inf)
    big = jnp.int32(2**30)
    acc_v = None
    for lo in range(0, _N_EMBED, _WINDOW):
        hi = min(lo + _WINDOW, _N_EMBED)
        mask = (iota >= lo) & (iota < hi)
        wv = jnp.max(jnp.where(mask, sim, neg), axis=1, keepdims=True)
        wi = jnp.min(jnp.where(mask & (sim == wv), iota, big),
                     axis=1, keepdims=True)
        if acc_v is None:
            acc_v, acc_i = wv, wi
        else:
            keep = acc_v > wv
            tie = (acc_v == wv) & (acc_i < wi)
            acc_i = jnp.where(keep | tie, acc_i, wi)
            acc_v = jnp.where(keep, acc_v, wv)
        acc_v = acc_v.astype(jnp.bfloat16).astype(jnp.float32)

    onehot = (iota == acc_i).astype(jnp.float32)
    zq = jax.lax.dot_general(onehot, e, (((1,), (0,)), ((), ())),
                             preferred_element_type=jnp.float32,
                             precision=jax.lax.Precision.HIGHEST)
    d = zq - z
    out_ref[...] = z + d
    idx_ref[0, 0, :] = acc_i[:, 0]
    rss_ref[0, 0, :] = jnp.sum(d * d, axis=1)


def kernel(z, embedding_weight):
    m_total = z.shape[0] * z.shape[1]
    n_blocks = m_total // _BM
    z2 = z.reshape(m_total, _E_DIM)
    out, idx, rss = pl.pallas_call(
        _vq_body,
        grid=(n_blocks,),
        in_specs=[
            pl.BlockSpec((_BM, _E_DIM), lambda i: (i, 0)),
            pl.BlockSpec((_N_EMBED, _E_DIM), lambda i: (0, 0)),
        ],
        out_specs=[
            pl.BlockSpec((_BM, _E_DIM), lambda i: (i, 0)),
            pl.BlockSpec((1, 1, _BM), lambda i: (i, 0, 0)),
            pl.BlockSpec((1, 1, _BM), lambda i: (i, 0, 0)),
        ],
        out_shape=[
            jax.ShapeDtypeStruct((m_total, _E_DIM), jnp.float32),
            jax.ShapeDtypeStruct((n_blocks, 1, _BM), jnp.int32),
            jax.ShapeDtypeStruct((n_blocks, 1, _BM), jnp.float32),
        ],
    )(z2, embedding_weight)
    total = jnp.sum(rss)
    vq_loss = total * jnp.float32(1.0 / (16384 * 256))
    commitment_loss = total * jnp.float32(0.25 / (16384 * 256))
    return (out.reshape(z.shape), vq_loss, commitment_loss,
            idx.reshape(m_total))


# speedup vs baseline: 1.3988x; 1.3988x over previous
import jax
import jax.numpy as jnp
from jax.experimental import pallas as pl
from jax.experimental.pallas import tpu as pltpu

_N_EMBED = 8192
_E_DIM = 256
_BM = 256
_WINDOW = 2736


def _vq_body(z_ref, e_ref, out_ref, idx_ref, rss_ref):
    z = z_ref[...]
    e = e_ref[...]
    zsq = jnp.sum(z * z, axis=1, keepdims=True)
    esq = jnp.sum(e * e, axis=1)[None, :]
    e_hi = e.astype(jnp.bfloat16)
    dot = jax.lax.dot_general(z.astype(jnp.bfloat16), e_hi,
                              (((1,), (1,)), ((), ())),
                              preferred_element_type=jnp.float32)
    sim = (zsq + esq) - 2.0 * dot

    iota = jax.lax.broadcasted_iota(jnp.int32, (1, _N_EMBED), 1)
    neg = jnp.float32(-jnp.inf)
    big = jnp.int32(2**30)
    acc_v = None
    for lo in range(0, _N_EMBED, _WINDOW):
        hi = min(lo + _WINDOW, _N_EMBED)
        mask = (iota >= lo) & (iota < hi)
        wsim = jnp.where(mask, sim, neg)
        wv = jnp.max(wsim, axis=1, keepdims=True)
        wi = jnp.min(jnp.where(wsim == wv, iota, big),
                     axis=1, keepdims=True)
        if acc_v is None:
            acc_v, acc_i = wv, wi
        else:
            keep = acc_v > wv
            tie = (acc_v == wv) & (acc_i < wi)
            acc_i = jnp.where(keep | tie, acc_i, wi)
            acc_v = jnp.where(keep, acc_v, wv)
        acc_v = acc_v.astype(jnp.bfloat16).astype(jnp.float32)

    onehot = (iota == acc_i).astype(jnp.float32).astype(jnp.bfloat16)
    e_lo = (e - e_hi.astype(jnp.float32)).astype(jnp.bfloat16)
    zq = (jax.lax.dot_general(onehot, e_hi, (((1,), (0,)), ((), ())),
                              preferred_element_type=jnp.float32)
          + jax.lax.dot_general(onehot, e_lo, (((1,), (0,)), ((), ())),
                                preferred_element_type=jnp.float32))
    d = zq - z
    out_ref[...] = z + d
    idx_ref[0, 0, :] = acc_i[:, 0]
    rss_ref[0, 0, :] = jnp.sum(d * d, axis=1)


def kernel(z, embedding_weight):
    m_total = z.shape[0] * z.shape[1]
    n_blocks = m_total // _BM
    z2 = z.reshape(m_total, _E_DIM)
    out, idx, rss = pl.pallas_call(
        _vq_body,
        grid=(n_blocks,),
        in_specs=[
            pl.BlockSpec((_BM, _E_DIM), lambda i: (i, 0)),
            pl.BlockSpec((_N_EMBED, _E_DIM), lambda i: (0, 0)),
        ],
        out_specs=[
            pl.BlockSpec((_BM, _E_DIM), lambda i: (i, 0)),
            pl.BlockSpec((1, 1, _BM), lambda i: (i, 0, 0)),
            pl.BlockSpec((1, 1, _BM), lambda i: (i, 0, 0)),
        ],
        out_shape=[
            jax.ShapeDtypeStruct((m_total, _E_DIM), jnp.float32),
            jax.ShapeDtypeStruct((n_blocks, 1, _BM), jnp.int32),
            jax.ShapeDtypeStruct((n_blocks, 1, _BM), jnp.float32),
        ],
        compiler_params=pltpu.CompilerParams(
            dimension_semantics=("parallel",)),
    )(z2, embedding_weight)
    total = jnp.sum(rss)
    vq_loss = total * jnp.float32(1.0 / (16384 * 256))
    commitment_loss = total * jnp.float32(0.25 / (16384 * 256))
    return (out.reshape(z.shape), vq_loss, commitment_loss,
            idx.reshape(m_total))


# speedup vs baseline: 1.7077x; 1.2208x over previous
import jax
import jax.numpy as jnp
from jax.experimental import pallas as pl
from jax.experimental.pallas import tpu as pltpu

_N_EMBED = 8192
_E_DIM = 256
_BM = 256
_WINDOW = 2736


def _vq_body(z_ref, e_ref, out_ref, idx_ref, rss_ref,
             ehi_ref, elo_ref, esq_ref):
    @pl.when(pl.program_id(0) == 0)
    def _prep():
        e = e_ref[...]
        ehi = e.astype(jnp.bfloat16)
        ehi_ref[...] = ehi
        elo_ref[...] = (e - ehi.astype(jnp.float32)).astype(jnp.bfloat16)
        esq_ref[...] = jnp.sum(e * e, axis=1)[None, :]

    z = z_ref[...]
    zb = z.astype(jnp.bfloat16)
    zsq = jnp.sum(z * z, axis=1, keepdims=True)
    big = jnp.int32(2**30)
    acc_v = None
    for lo in range(0, _N_EMBED, _WINDOW):
        hi = min(lo + _WINDOW, _N_EMBED)
        dot = jax.lax.dot_general(zb, ehi_ref[lo:hi, :],
                                  (((1,), (1,)), ((), ())),
                                  preferred_element_type=jnp.float32)
        sim = (zsq + esq_ref[:, lo:hi]) - 2.0 * dot
        iota = jax.lax.broadcasted_iota(jnp.int32, (1, hi - lo), 1) + lo
        wv = jnp.max(sim, axis=1, keepdims=True)
        wi = jnp.min(jnp.where(sim == wv, iota, big),
                     axis=1, keepdims=True)
        if acc_v is None:
            acc_v, acc_i = wv, wi
        else:
            keep = acc_v > wv
            tie = (acc_v == wv) & (acc_i < wi)
            acc_i = jnp.where(keep | tie, acc_i, wi)
            acc_v = jnp.where(keep, acc_v, wv)
        acc_v = acc_v.astype(jnp.bfloat16).astype(jnp.float32)

    iota_full = jax.lax.broadcasted_iota(jnp.int32, (1, _N_EMBED), 1)
    onehot = (iota_full == acc_i).astype(jnp.float32).astype(jnp.bfloat16)
    zq = (jax.lax.dot_general(onehot, ehi_ref[...], (((1,), (0,)), ((), ())),
                              preferred_element_type=jnp.float32)
          + jax.lax.dot_general(onehot, elo_ref[...], (((1,), (0,)), ((), ())),
                                preferred_element_type=jnp.float32))
    d = zq - z
    out_ref[...] = z + d
    idx_ref[0, 0, :] = acc_i[:, 0]
    rss_ref[0, 0, :] = jnp.sum(d * d, axis=1)


def kernel(z, embedding_weight):
    m_total = z.shape[0] * z.shape[1]
    n_blocks = m_total // _BM
    z2 = z.reshape(m_total, _E_DIM)
    out, idx, rss = pl.pallas_call(
        _vq_body,
        grid=(n_blocks,),
        in_specs=[
            pl.BlockSpec((_BM, _E_DIM), lambda i: (i, 0)),
            pl.BlockSpec((_N_EMBED, _E_DIM), lambda i: (0, 0)),
        ],
        out_specs=[
            pl.BlockSpec((_BM, _E_DIM), lambda i: (i, 0)),
            pl.BlockSpec((1, 1, _BM), lambda i: (i, 0, 0)),
            pl.BlockSpec((1, 1, _BM), lambda i: (i, 0, 0)),
        ],
        out_shape=[
            jax.ShapeDtypeStruct((m_total, _E_DIM), jnp.float32),
            jax.ShapeDtypeStruct((n_blocks, 1, _BM), jnp.int32),
            jax.ShapeDtypeStruct((n_blocks, 1, _BM), jnp.float32),
        ],
        scratch_shapes=[
            pltpu.VMEM((_N_EMBED, _E_DIM), jnp.bfloat16),
            pltpu.VMEM((_N_EMBED, _E_DIM), jnp.bfloat16),
            pltpu.VMEM((1, _N_EMBED), jnp.float32),
        ],
    )(z2, embedding_weight)
    total = jnp.sum(rss)
    vq_loss = total * jnp.float32(1.0 / (16384 * 256))
    commitment_loss = total * jnp.float32(0.25 / (16384 * 256))
    return (out.reshape(z.shape), vq_loss, commitment_loss,
            idx.reshape(m_total))
